# baseline (speedup 1.0000x reference)
import functools

import jax
import jax.numpy as jnp
from jax.experimental import pallas as pl
from jax.experimental.pallas import tpu as pltpu

HIDDEN = 1024
INTER = 2816
K_ACTIVE = 704
ROW_TILE = 256


def _fused_kernel(x_ref, wg_ref, wu_ref, wd_ref, out_ref, az_ref, k16_ref):
    x = x_ref[...].astype(jnp.bfloat16)

    nt = (((1,), (1,)), ((), ()))
    g = jax.lax.dot_general(x, wg_ref[...], nt, preferred_element_type=jnp.float32)
    u = jax.lax.dot_general(x, wu_ref[...], nt, preferred_element_type=jnp.float32)
    z = (g * jax.nn.sigmoid(g)) * u
    az_ref[...] = jnp.abs(z)
    zbits = jax.lax.bitcast_convert_type(z, jnp.int32) & jnp.int32(0x7FFFFFFF)
    k16_ref[...] = (zbits >> 16).astype(jnp.int16)

    Q = ROW_TILE // 4

    def body16(i, carry):
        bit = jnp.int32(1) << (jnp.int32(30) - i)
        out = []
        for c, res in enumerate(carry):
            cand = res | bit
            c16 = (cand >> 16).astype(jnp.int16)
            pred = k16_ref[c * Q:(c + 1) * Q, :] >= c16
            ones = jnp.where(pred, jnp.int16(1), jnp.int16(0))
            cnt = jnp.sum(ones, axis=1, keepdims=True, dtype=jnp.int32)
            out.append(jnp.where(cnt >= K_ACTIVE, cand, res))
        return tuple(out)

    def body(i, carry):
        bit = jnp.int32(1) << (jnp.int32(15) - i)
        out = []
        for c, res in enumerate(carry):
            cand = res | bit
            candf = jax.lax.bitcast_convert_type(cand, jnp.float32)
            ones = jnp.where(az_ref[c * Q:(c + 1) * Q, :] >= candf, 1.0, 0.0)
            cnt = jnp.sum(ones, axis=1, keepdims=True)
            out.append(jnp.where(cnt >= float(K_ACTIVE), cand, res))
        return tuple(out)

    res0 = jnp.zeros((Q, 1), jnp.int32)
    carry = jax.lax.fori_loop(0, 15, body16, (res0,) * 4)
    carry = jax.lax.fori_loop(0, 7, body, carry)
    res = jnp.concatenate(carry, axis=0)

    thresh = jax.lax.bitcast_convert_type(res, jnp.float32)
    zm = jnp.where(az_ref[...] >= thresh, z, 0.0).astype(jnp.bfloat16)
    out_ref[...] = jax.lax.dot_general(
        zm, wd_ref[...], nt, preferred_element_type=jnp.float32
    )


@jax.jit
def kernel(x, Wg, Wu, Wd):
    B, S, H = x.shape
    rows = B * S
    xf = x.reshape(rows, H)

    out = pl.pallas_call(
        _fused_kernel,
        grid=(rows // ROW_TILE,),
        in_specs=[
            pl.BlockSpec((ROW_TILE, HIDDEN), lambda i: (i, 0)),
            pl.BlockSpec((INTER, HIDDEN), lambda i: (0, 0)),
            pl.BlockSpec((INTER, HIDDEN), lambda i: (0, 0)),
            pl.BlockSpec((HIDDEN, INTER), lambda i: (0, 0)),
        ],
        out_specs=pl.BlockSpec((ROW_TILE, HIDDEN), lambda i: (i, 0)),
        out_shape=jax.ShapeDtypeStruct((rows, HIDDEN), jnp.float32),
        scratch_shapes=[
            pltpu.VMEM((ROW_TILE, INTER), jnp.float32),
            pltpu.VMEM((ROW_TILE, INTER), jnp.int16),
        ],
        compiler_params=pltpu.CompilerParams(
            dimension_semantics=("arbitrary",),
        ),
    )(
        xf,
        Wg.astype(jnp.bfloat16),
        Wu.astype(jnp.bfloat16),
        Wd.astype(jnp.bfloat16),
    )
    return out.reshape(B, S, H)


# speedup vs baseline: 1.0495x; 1.0495x over previous
import functools

import jax
import jax.numpy as jnp
from jax.experimental import pallas as pl
from jax.experimental.pallas import tpu as pltpu

HIDDEN = 1024
INTER = 2816
K_ACTIVE = 704
ROW_TILE = 256


def _fused_kernel(x_ref, wg_ref, wu_ref, wd_ref, out_ref, az_ref):
    x = x_ref[...].astype(jnp.bfloat16)

    nt = (((1,), (1,)), ((), ()))
    g = jax.lax.dot_general(x, wg_ref[...], nt, preferred_element_type=jnp.float32)
    u = jax.lax.dot_general(x, wu_ref[...], nt, preferred_element_type=jnp.float32)
    z = (g * jax.nn.sigmoid(g)) * u
    az_ref[...] = jnp.abs(z)

    Q = ROW_TILE // 4

    def body(i, carry):
        bit = jnp.int32(1) << (jnp.int32(30) - i)
        out = []
        for c, res in enumerate(carry):
            cand = res | bit
            candf = jax.lax.bitcast_convert_type(cand, jnp.float32)
            ones = jnp.where(az_ref[c * Q:(c + 1) * Q, :] >= candf, 1.0, 0.0)
            cnt = jnp.sum(ones, axis=1, keepdims=True)
            out.append(jnp.where(cnt >= float(K_ACTIVE), cand, res))
        return tuple(out)

    res0 = jnp.zeros((Q, 1), jnp.int32)
    carry = jax.lax.fori_loop(0, 22, body, (res0,) * 4)
    res = jnp.concatenate(carry, axis=0)

    thresh = jax.lax.bitcast_convert_type(res, jnp.float32)
    zm = jnp.where(az_ref[...] >= thresh, z, 0.0).astype(jnp.bfloat16)
    out_ref[...] = jax.lax.dot_general(
        zm, wd_ref[...], nt, preferred_element_type=jnp.float32
    )


@jax.jit
def kernel(x, Wg, Wu, Wd):
    B, S, H = x.shape
    rows = B * S
    xf = x.reshape(rows, H)

    out = pl.pallas_call(
        _fused_kernel,
        grid=(rows // ROW_TILE,),
        in_specs=[
            pl.BlockSpec((ROW_TILE, HIDDEN), lambda i: (i, 0)),
            pl.BlockSpec((INTER, HIDDEN), lambda i: (0, 0)),
            pl.BlockSpec((INTER, HIDDEN), lambda i: (0, 0)),
            pl.BlockSpec((HIDDEN, INTER), lambda i: (0, 0)),
        ],
        out_specs=pl.BlockSpec((ROW_TILE, HIDDEN), lambda i: (i, 0)),
        out_shape=jax.ShapeDtypeStruct((rows, HIDDEN), jnp.float32),
        scratch_shapes=[pltpu.VMEM((ROW_TILE, INTER), jnp.float32)],
        compiler_params=pltpu.CompilerParams(
            dimension_semantics=("arbitrary",),
        ),
    )(
        xf,
        Wg.astype(jnp.bfloat16),
        Wu.astype(jnp.bfloat16),
        Wd.astype(jnp.bfloat16),
    )
    return out.reshape(B, S, H)


# speedup vs baseline: 1.1221x; 1.0692x over previous
import functools

import jax
import jax.numpy as jnp
from jax.experimental import pallas as pl
from jax.experimental.pallas import tpu as pltpu

HIDDEN = 1024
INTER = 2816
K_ACTIVE = 704
ROW_TILE = 512


def _fused_kernel(x_ref, wg_ref, wu_ref, wd_ref, out_ref, az_ref):
    x = x_ref[...].astype(jnp.bfloat16)

    nt = (((1,), (1,)), ((), ()))
    g = jax.lax.dot_general(x, wg_ref[...], nt, preferred_element_type=jnp.float32)
    u = jax.lax.dot_general(x, wu_ref[...], nt, preferred_element_type=jnp.float32)
    z = (g * jax.nn.sigmoid(g)) * u
    az_ref[...] = jnp.abs(z)

    Q = ROW_TILE // 4

    def body(i, carry):
        bit = jnp.int32(1) << (jnp.int32(30) - i)
        out = []
        for c, res in enumerate(carry):
            cand = res | bit
            candf = jax.lax.bitcast_convert_type(cand, jnp.float32)
            ones = jnp.where(az_ref[c * Q:(c + 1) * Q, :] >= candf, 1.0, 0.0)
            cnt = jnp.sum(ones, axis=1, keepdims=True)
            out.append(jnp.where(cnt >= float(K_ACTIVE), cand, res))
        return tuple(out)

    res0 = jnp.zeros((Q, 1), jnp.int32)
    carry = jax.lax.fori_loop(0, 22, body, (res0,) * 4)
    res = jnp.concatenate(carry, axis=0)

    thresh = jax.lax.bitcast_convert_type(res, jnp.float32)
    zm = jnp.where(az_ref[...] >= thresh, z, 0.0).astype(jnp.bfloat16)
    out_ref[...] = jax.lax.dot_general(
        zm, wd_ref[...], nt, preferred_element_type=jnp.float32
    )


@jax.jit
def kernel(x, Wg, Wu, Wd):
    B, S, H = x.shape
    rows = B * S
    xf = x.reshape(rows, H)

    out = pl.pallas_call(
        _fused_kernel,
        grid=(rows // ROW_TILE,),
        in_specs=[
            pl.BlockSpec((ROW_TILE, HIDDEN), lambda i: (i, 0)),
            pl.BlockSpec((INTER, HIDDEN), lambda i: (0, 0)),
            pl.BlockSpec((INTER, HIDDEN), lambda i: (0, 0)),
            pl.BlockSpec((HIDDEN, INTER), lambda i: (0, 0)),
        ],
        out_specs=pl.BlockSpec((ROW_TILE, HIDDEN), lambda i: (i, 0)),
        out_shape=jax.ShapeDtypeStruct((rows, HIDDEN), jnp.float32),
        scratch_shapes=[pltpu.VMEM((ROW_TILE, INTER), jnp.float32)],
        compiler_params=pltpu.CompilerParams(
            dimension_semantics=("arbitrary",),
        ),
    )(
        xf,
        Wg.astype(jnp.bfloat16),
        Wu.astype(jnp.bfloat16),
        Wd.astype(jnp.bfloat16),
    )
    return out.reshape(B, S, H)


# speedup vs baseline: 1.1729x; 1.0453x over previous
import functools

import jax
import jax.numpy as jnp
from jax.experimental import pallas as pl
from jax.experimental.pallas import tpu as pltpu

HIDDEN = 1024
INTER = 2816
K_ACTIVE = 704
ROW_TILE = 1024


def _fused_kernel(x_ref, wg_ref, wu_ref, wd_ref, out_ref, az_ref):
    x = x_ref[...].astype(jnp.bfloat16)

    nt = (((1,), (1,)), ((), ()))
    g = jax.lax.dot_general(x, wg_ref[...], nt, preferred_element_type=jnp.float32)
    u = jax.lax.dot_general(x, wu_ref[...], nt, preferred_element_type=jnp.float32)
    z = (g * jax.nn.sigmoid(g)) * u
    az_ref[...] = jnp.abs(z)

    Q = ROW_TILE // 4

    def body(i, carry):
        bit = jnp.int32(1) << (jnp.int32(30) - i)
        out = []
        for c, res in enumerate(carry):
            cand = res | bit
            candf = jax.lax.bitcast_convert_type(cand, jnp.float32)
            ones = jnp.where(az_ref[c * Q:(c + 1) * Q, :] >= candf, 1.0, 0.0)
            cnt = jnp.sum(ones, axis=1, keepdims=True)
            out.append(jnp.where(cnt >= float(K_ACTIVE), cand, res))
        return tuple(out)

    res0 = jnp.zeros((Q, 1), jnp.int32)
    carry = jax.lax.fori_loop(0, 22, body, (res0,) * 4)
    res = jnp.concatenate(carry, axis=0)

    thresh = jax.lax.bitcast_convert_type(res, jnp.float32)
    zm = jnp.where(az_ref[...] >= thresh, z, 0.0).astype(jnp.bfloat16)
    out_ref[...] = jax.lax.dot_general(
        zm, wd_ref[...], nt, preferred_element_type=jnp.float32
    )


@jax.jit
def kernel(x, Wg, Wu, Wd):
    B, S, H = x.shape
    rows = B * S
    xf = x.reshape(rows, H)

    out = pl.pallas_call(
        _fused_kernel,
        grid=(rows // ROW_TILE,),
        in_specs=[
            pl.BlockSpec((ROW_TILE, HIDDEN), lambda i: (i, 0)),
            pl.BlockSpec((INTER, HIDDEN), lambda i: (0, 0)),
            pl.BlockSpec((INTER, HIDDEN), lambda i: (0, 0)),
            pl.BlockSpec((HIDDEN, INTER), lambda i: (0, 0)),
        ],
        out_specs=pl.BlockSpec((ROW_TILE, HIDDEN), lambda i: (i, 0)),
        out_shape=jax.ShapeDtypeStruct((rows, HIDDEN), jnp.float32),
        scratch_shapes=[pltpu.VMEM((ROW_TILE, INTER), jnp.float32)],
        compiler_params=pltpu.CompilerParams(
            dimension_semantics=("arbitrary",),
        ),
    )(
        xf,
        Wg.astype(jnp.bfloat16),
        Wu.astype(jnp.bfloat16),
        Wd.astype(jnp.bfloat16),
    )
    return out.reshape(B, S, H)
